# baseline (speedup 1.0000x reference)
import functools

import jax
import jax.numpy as jnp
from jax import lax
from jax.experimental import pallas as pl
from jax.experimental.pallas import tpu as pltpu
from jax.experimental.pallas import tpu_sc as plsc

NC = 2
NS = 16
NW = NC * NS
NBUF = 8


def _make_sc_gather(nb, h, d):
  rows_pw = nb // NW
  mesh = plsc.VectorSubcoreMesh(core_axis_name="c", subcore_axis_name="s")

  @functools.partial(
      pl.kernel,
      mesh=mesh,
      compiler_params=pltpu.CompilerParams(use_tc_tiling_on_sc=False),
      out_type=jax.ShapeDtypeStruct((nb, h, d), jnp.float32),
      scratch_types=[
          pltpu.VMEM((rows_pw, h), jnp.int32),
          pltpu.VMEM((NBUF, h, d), jnp.float32),
          pltpu.SemaphoreType.DMA((NBUF,)),
          pltpu.SemaphoreType.DMA((NBUF,)),
      ],
  )
  def sc_gather(idx_hbm, table_hbm, out_hbm, idx_v, rows_v, gsem, osem):
    wid = lax.axis_index("s") * NC + lax.axis_index("c")
    base = wid * rows_pw
    pltpu.sync_copy(idx_hbm.at[wid], idx_v)

    def g_start(j, b):
      pltpu.async_copy(table_hbm.at[idx_v.at[j]], rows_v.at[b], gsem.at[b])

    def g_wait(j, b):
      pltpu.make_async_copy(
          table_hbm.at[idx_v.at[j]], rows_v.at[b], gsem.at[b]).wait()

    def w_start(j, b):
      pltpu.async_copy(rows_v.at[b], out_hbm.at[base + j], osem.at[b])

    def w_wait(j, b):
      pltpu.make_async_copy(
          rows_v.at[b], out_hbm.at[base + j], osem.at[b]).wait()

    for b in range(NBUF):
      g_start(b, b)

    def body(j0, _):
      for b in range(NBUF):
        j = j0 + b
        g_wait(j, b)
        w_start(j, b)
        w_wait(j, b)
        g_start(j + NBUF, b)
      return ()

    lax.fori_loop(0, (rows_pw - NBUF) // NBUF,
                  lambda i, c: body(i * NBUF, c), (), unroll=False)

    for b in range(NBUF):
      j = rows_pw - NBUF + b
      g_wait(j, b)
      w_start(j, b)
    for b in range(NBUF):
      j = rows_pw - NBUF + b
      w_wait(j, b)

  return sc_gather


def kernel(inp, table):
  nb, h = inp.shape
  v, d = table.shape
  assert nb % (NW * NBUF) == 0
  idx = inp.reshape(NW, nb // NW, h).astype(jnp.int32)
  fn = _make_sc_gather(nb, h, d)
  return fn(idx, table)


# speedup vs baseline: 1.2403x; 1.2403x over previous
import functools

import jax
import jax.numpy as jnp
from jax import lax
from jax.experimental import pallas as pl
from jax.experimental.pallas import tpu as pltpu
from jax.experimental.pallas import tpu_sc as plsc

NC = 2
NS = 16
NW = NC * NS
NBUF = 8
DP = 128


def _make_sc_gather(nb, h, d):
  rows_pw = nb // NW
  mesh = plsc.VectorSubcoreMesh(core_axis_name="c", subcore_axis_name="s")

  @functools.partial(
      pl.kernel,
      mesh=mesh,
      compiler_params=pltpu.CompilerParams(use_tc_tiling_on_sc=True),
      out_type=jax.ShapeDtypeStruct((nb, h, DP), jnp.float32),
      scratch_types=[
          pltpu.VMEM((rows_pw, h), jnp.int32),
          pltpu.VMEM((NBUF, h, DP), jnp.float32),
          pltpu.SemaphoreType.DMA((NBUF,)),
          pltpu.SemaphoreType.DMA((NBUF,)),
      ],
  )
  def sc_gather(idx_hbm, table_hbm, out_hbm, idx_v, rows_v, gsem, osem):
    wid = lax.axis_index("s") * NC + lax.axis_index("c")
    base = wid * rows_pw
    pltpu.sync_copy(idx_hbm.at[wid], idx_v)

    def g_start(j, b):
      pltpu.async_copy(table_hbm.at[idx_v.at[j]], rows_v.at[b], gsem.at[b])

    def g_wait(j, b):
      pltpu.make_async_copy(
          table_hbm.at[idx_v.at[j]], rows_v.at[b], gsem.at[b]).wait()

    def w_start(j, b):
      pltpu.async_copy(rows_v.at[b], out_hbm.at[base + j], osem.at[b])

    def w_wait(j, b):
      pltpu.make_async_copy(
          rows_v.at[b], out_hbm.at[base + j], osem.at[b]).wait()

    for b in range(NBUF):
      g_start(b, b)

    def body(j0, _):
      for b in range(NBUF):
        j = j0 + b
        g_wait(j, b)
        w_start(j, b)
        w_wait(j, b)
        g_start(j + NBUF, b)
      return ()

    lax.fori_loop(0, (rows_pw - NBUF) // NBUF,
                  lambda i, c: body(i * NBUF, c), (), unroll=False)

    for b in range(NBUF):
      j = rows_pw - NBUF + b
      g_wait(j, b)
      w_start(j, b)
    for b in range(NBUF):
      j = rows_pw - NBUF + b
      w_wait(j, b)

  return sc_gather


def kernel(inp, table):
  nb, h = inp.shape
  v, d = table.shape
  assert nb % (NW * NBUF) == 0
  table_p = jnp.pad(table, ((0, 0), (0, DP - d)))
  idx = inp.reshape(NW, nb // NW, h).astype(jnp.int32)
  fn = _make_sc_gather(nb, h, d)
  return fn(idx, table_p)[:, :, :d]
